# baseline (speedup 1.0000x reference)
import functools

import jax
import jax.numpy as jnp
from jax import lax
from jax.experimental import pallas as pl
from jax.experimental.pallas import tpu as pltpu
from jax.experimental.pallas import tpu_sc as plsc

B = 4096
L = 50
D = 64
VOCAB_ROWS = 100000
LANES = 16
NC = 2
NS = 16
NW = NC * NS
ROWS_PER_W = B // NW
CHUNK_ROWS = 8
IDX_RAW = CHUNK_ROWS * L
IDX_PAD = 400
CHUNKS_PER_W = ROWS_PER_W // CHUNK_ROWS
NBUF = 2

BLKC = 512
NBLK = -(-(VOCAB_ROWS // 2) // BLKC)
VP = NBLK * BLKC


def _tc_relayout_body(xlo_ref, xhi_ref, o_ref):
  o_ref[...] = jnp.concatenate([xlo_ref[...], xhi_ref[...]], axis=1)


_tc_relayout = pl.pallas_call(
    _tc_relayout_body,
    grid=(NBLK,),
    in_specs=[
        pl.BlockSpec((BLKC, D), lambda j: (j, 0)),
        pl.BlockSpec((BLKC, D), lambda j: (j + NBLK, 0)),
    ],
    out_specs=pl.BlockSpec((BLKC, 2 * D), lambda j: (j, 0)),
    out_shape=jax.ShapeDtypeStruct((VP, 2 * D), jnp.float32),
)


def _worker_body(idx_hbm, table_hbm, out_hbm, idx_v, rows, out_v, sems):
  wid = lax.axis_index("s") * NC + lax.axis_index("c")
  base_chunk = wid * CHUNKS_PER_W
  pltpu.sync_copy(idx_hbm.at[pl.ds(base_chunk, CHUNKS_PER_W)], idx_v)

  def gather(j, b):
    return pltpu.make_async_copy(table_hbm.at[idx_v.at[j]], rows[b], sems[b])

  for b in range(NBUF):
    gather(b, b).start()

  def step(p, carry):
    for b in range(NBUF):
      j = p * NBUF + b
      gather(j, b).wait()
      buf = rows[b]

      def row_body(r, carry2):
        base = r * L
        for d in range(D // LANES):
          acc = buf[base, pl.ds(d * LANES, LANES)]
          for s in range(1, L):
            acc = jnp.maximum(acc, buf[base + s, pl.ds(d * LANES, LANES)])
          out_v[j * CHUNK_ROWS + r, pl.ds(d * LANES, LANES)] = acc
        return carry2

      lax.fori_loop(0, CHUNK_ROWS, row_body, None)

      nxt = j + NBUF

      @pl.when(nxt < CHUNKS_PER_W)
      def _():
        gather(nxt, b).start()
    return carry

  lax.fori_loop(0, CHUNKS_PER_W // NBUF, step, None)
  pltpu.sync_copy(out_v, out_hbm.at[pl.ds(wid * ROWS_PER_W, ROWS_PER_W)])


@functools.partial(
    pl.kernel,
    out_type=jax.ShapeDtypeStruct((B, D), jnp.float32),
    mesh=plsc.VectorSubcoreMesh(core_axis_name="c", subcore_axis_name="s"),
    scratch_types=[
        pltpu.VMEM((CHUNKS_PER_W, IDX_PAD), jnp.int32),
        [pltpu.VMEM((IDX_PAD, D), jnp.float32) for _ in range(NBUF)],
        pltpu.VMEM((ROWS_PER_W, D), jnp.float32),
        [pltpu.SemaphoreType.DMA for _ in range(NBUF)],
    ],
    compiler_params=pltpu.CompilerParams(
        use_tc_tiling_on_sc=False, needs_layout_passes=False),
)
def _sc_embed_maxpool(idx_hbm, table_hbm, out_hbm, idx_v, rows, out_v, sems):
  _worker_body(idx_hbm, table_hbm, out_hbm, idx_v, rows, out_v, sems)


def kernel(char_ids, table):
  v = char_ids.astype(jnp.int32)
  idx = jnp.where(v < VP, 2 * v, 2 * (v - VP) + 1)
  idx = idx.reshape(NW * CHUNKS_PER_W, IDX_RAW)
  if IDX_PAD > IDX_RAW:
    idx = jnp.pad(idx, ((0, 0), (0, IDX_PAD - IDX_RAW)))
  packed = _tc_relayout(table, table)
  table_lin = packed.reshape(2 * VP, D)
  return _sc_embed_maxpool(idx, table_lin)


# speedup vs baseline: 1.3948x; 1.3948x over previous
import functools

import jax
import jax.numpy as jnp
from jax import lax
from jax.experimental import pallas as pl
from jax.experimental.pallas import tpu as pltpu
from jax.experimental.pallas import tpu_sc as plsc

B = 4096
L = 50
D = 64
LANES = 16
NC = 2
NS = 16
NW = NC * NS
ROWS_PER_W = B // NW
CHUNK_ROWS = 16
IDX_RAW = CHUNK_ROWS * L
IDX_PAD = 800
CHUNKS_PER_W = ROWS_PER_W // CHUNK_ROWS
NBUF = 2


def _worker_body(idx_hbm, table_hbm, out_hbm, idx_v, rows, out_v, sems):
  wid = lax.axis_index("s") * NC + lax.axis_index("c")
  base_chunk = wid * CHUNKS_PER_W
  pltpu.sync_copy(idx_hbm.at[pl.ds(base_chunk, CHUNKS_PER_W)], idx_v)
  lane = lax.iota(jnp.int32, LANES)

  def gather(j, b):
    return pltpu.make_async_copy(table_hbm.at[idx_v.at[j]], rows[b], sems[b])

  for b in range(NBUF):
    gather(b, b).start()

  def step(p, carry):
    for b in range(NBUF):
      j = p * NBUF + b
      gather(j, b).wait()
      buf = rows[b]

      def row_body(r, carry2):
        base = r * L
        col = jnp.full((LANES,), j * CHUNK_ROWS + r, jnp.int32)
        for d in range(D // LANES):
          acc = buf[base, pl.ds(d * LANES, LANES)]
          for s in range(1, L):
            acc = jnp.maximum(acc, buf[base + s, pl.ds(d * LANES, LANES)])
          plsc.store_scatter(out_v, [lane + d * LANES, col], acc)
        return carry2

      lax.fori_loop(0, CHUNK_ROWS, row_body, None)

      nxt = j + NBUF

      @pl.when(nxt < CHUNKS_PER_W)
      def _():
        gather(nxt, b).start()
    return carry

  lax.fori_loop(0, CHUNKS_PER_W // NBUF, step, None)
  pltpu.sync_copy(out_v, out_hbm.at[:, pl.ds(wid * ROWS_PER_W, ROWS_PER_W)])


@functools.partial(
    pl.kernel,
    out_type=jax.ShapeDtypeStruct((D, B), jnp.float32),
    mesh=plsc.VectorSubcoreMesh(core_axis_name="c", subcore_axis_name="s"),
    scratch_types=[
        pltpu.VMEM((CHUNKS_PER_W, IDX_PAD), jnp.int32),
        [pltpu.VMEM((IDX_PAD, D), jnp.float32) for _ in range(NBUF)],
        pltpu.VMEM((D, ROWS_PER_W), jnp.float32),
        [pltpu.SemaphoreType.DMA for _ in range(NBUF)],
    ],
    compiler_params=pltpu.CompilerParams(
        use_tc_tiling_on_sc=False, needs_layout_passes=False),
)
def _sc_embed_maxpool(idx_hbm, table_hbm, out_hbm, idx_v, rows, out_v, sems):
  _worker_body(idx_hbm, table_hbm, out_hbm, idx_v, rows, out_v, sems)


def kernel(char_ids, table):
  idx = char_ids.astype(jnp.int32).reshape(NW * CHUNKS_PER_W, IDX_RAW)
  if IDX_PAD > IDX_RAW:
    idx = jnp.pad(idx, ((0, 0), (0, IDX_PAD - IDX_RAW)))
  out_t = _sc_embed_maxpool(idx, table)
  return out_t.T


# speedup vs baseline: 1.5027x; 1.0773x over previous
import functools

import jax
import jax.numpy as jnp
from jax import lax
from jax.experimental import pallas as pl
from jax.experimental.pallas import tpu as pltpu
from jax.experimental.pallas import tpu_sc as plsc

B = 4096
L = 50
D = 64
LANES = 16
NC = 2
NS = 16
NW = NC * NS
ROWS_PER_W = B // NW
CHUNK_ROWS = 16
IDX_RAW = CHUNK_ROWS * L
IDX_PAD = 800
CHUNKS_PER_W = ROWS_PER_W // CHUNK_ROWS
NBUF = 2


def _worker_body(idx_hbm, table_hbm, out_hbm, idx_v, rows, out_v, sems):
  wid = lax.axis_index("s") * NC + lax.axis_index("c")
  base_chunk = wid * CHUNKS_PER_W
  pltpu.sync_copy(idx_hbm.at[pl.ds(base_chunk, CHUNKS_PER_W)], idx_v)
  lane = lax.iota(jnp.int32, LANES)

  def gather(j, b):
    return pltpu.make_async_copy(table_hbm.at[idx_v.at[j]], rows[b], sems[b])

  for b in range(NBUF):
    gather(b, b).start()

  def step(p, carry):
    for b in range(NBUF):
      j = p * NBUF + b
      gather(j, b).wait()
      buf = rows[b]

      def row_body(r, carry2):
        base = r * L
        col = jnp.full((LANES,), j * CHUNK_ROWS + r, jnp.int32)
        for d in range(D // LANES):
          acc = buf[base, pl.ds(d * LANES, LANES)]
          for s in range(1, L):
            acc = jnp.maximum(acc, buf[base + s, pl.ds(d * LANES, LANES)])
          plsc.store_scatter(out_v, [lane + d * LANES, col], acc)
        return carry2

      lax.fori_loop(0, CHUNK_ROWS, row_body, None)

      nxt = j + NBUF

      @pl.when(nxt < CHUNKS_PER_W)
      def _():
        gather(nxt, b).start()
    return carry

  lax.fori_loop(0, CHUNKS_PER_W // NBUF, step, None)
  pltpu.sync_copy(out_v, out_hbm.at[:, pl.ds(wid * ROWS_PER_W, ROWS_PER_W)])


@functools.partial(
    pl.kernel,
    out_type=jax.ShapeDtypeStruct((D, B), jnp.float32),
    mesh=plsc.VectorSubcoreMesh(core_axis_name="c", subcore_axis_name="s"),
    scratch_types=[
        pltpu.VMEM((CHUNKS_PER_W, IDX_PAD), jnp.int32),
        [pltpu.VMEM((IDX_PAD, D), jnp.float32) for _ in range(NBUF)],
        pltpu.VMEM((D, ROWS_PER_W), jnp.float32),
        [pltpu.SemaphoreType.DMA for _ in range(NBUF)],
    ],
    compiler_params=pltpu.CompilerParams(
        use_tc_tiling_on_sc=False, needs_layout_passes=False),
)
def _sc_embed_maxpool(idx_hbm, table_hbm, out_hbm, idx_v, rows, out_v, sems):
  _worker_body(idx_hbm, table_hbm, out_hbm, idx_v, rows, out_v, sems)


def kernel(char_ids, table):
  idx = 2 * char_ids.astype(jnp.int32).reshape(NW * CHUNKS_PER_W, IDX_RAW)
  if IDX_PAD > IDX_RAW:
    idx = jnp.pad(idx, ((0, 0), (0, IDX_PAD - IDX_RAW)))
  vocab = table.shape[0]
  tab128 = jnp.pad(table, ((0, 0), (0, D)))
  tab_lin = tab128.reshape(2 * vocab, D)
  out_t = _sc_embed_maxpool(idx, tab_lin)
  return out_t.T
